# baseline (speedup 1.0000x reference)
import functools

import jax
import jax.numpy as jnp
from jax import lax
from jax.experimental import pallas as pl
from jax.experimental.pallas import tpu as pltpu
from jax.experimental.pallas import tpu_sc as plsc

N_NODES = 10000
N_EDGES = 160000
D_IN = 256
D_HID = 512
D_OUT = 256

NC = 2
NS = 16
Q = 64
CW = 16
CH = 125
EPT = N_EDGES // NS
NCHUNK = EPT // CH
NPAD = 10240
ROWS_PT = NPAD // NS

_sc_mesh = plsc.VectorSubcoreMesh(
    core_axis_name="c", subcore_axis_name="s", num_cores=NC, num_subcores=NS
)
_sc_params = pltpu.CompilerParams(use_tc_tiling_on_sc=False)


def _fire_gather(tab, src_v, buf, gsem, c):
    pltpu.async_copy(tab.at[src_v.at[c]], buf, gsem)


def _drain_gather(tab, src_v, buf, gsem, c):
    pltpu.make_async_copy(tab.at[src_v.at[c]], buf, gsem).wait()


def _fire_scatter(acc, dst_v, buf, ssem, c):
    pltpu.async_copy(buf, acc.at[dst_v.at[c]], ssem, add=True)


def _drain_scatter(acc, dst_v, buf, ssem, c):
    pltpu.make_async_copy(buf, acc.at[dst_v.at[c]], ssem).wait()


NBUF = 5


def _edge_steps(tab, src_v, dst_v, bufs, gsem, ssem, acc, cnt=None):
    def fire_cnt(c):
        if cnt is None:
            return
        ones_v, cntacc, csem, cid = cnt

        @pl.when(cid == (c % 2))
        def _():
            pltpu.async_copy(ones_v, cntacc.at[dst_v.at[c]], csem, add=True)

    for j in range(3):
        _fire_gather(tab, src_v, bufs[j], gsem, j)
    for j in range(NBUF):
        _drain_gather(tab, src_v, bufs[j], gsem, j)
        _fire_scatter(acc, dst_v, bufs[j], ssem, j)
        fire_cnt(j)
        if j >= 2:
            _drain_scatter(acc, dst_v, bufs[j - 2], ssem, j - 2)
        _fire_gather(tab, src_v, bufs[(j + 3) % NBUF], gsem, j + 3)

    def body(i, carry):
        c0 = NBUF * i
        for j in range(NBUF):
            c = c0 + j
            _drain_gather(tab, src_v, bufs[j], gsem, c)
            _fire_scatter(acc, dst_v, bufs[j], ssem, c)
            fire_cnt(c)
            _drain_scatter(acc, dst_v, bufs[(j + 3) % NBUF], ssem, c - 2)
            cg = jnp.minimum(c + 3, NCHUNK - 1)
            _fire_gather(tab, src_v, bufs[(j + 3) % NBUF], gsem, cg)
        return carry

    lax.fori_loop(1, NCHUNK // NBUF, body, 0)

    for j in range(2):
        c = NCHUNK - 2 + j
        _drain_scatter(acc, dst_v, bufs[c % NBUF], ssem, c)
    for j in range(3):
        _drain_gather(tab, src_v, bufs[j], gsem, NCHUNK - 1)
    if cnt is not None:
        ones_v, cntacc, csem, cid = cnt

        def cdrain(t, carry):
            pltpu.make_async_copy(ones_v, cntacc.at[dst_v.at[0]], csem).wait()
            return carry

        lax.fori_loop(0, NCHUNK // 2, cdrain, 0)


def _sc_pass(tab_c0, tab_c1, out_c0, out_c1, z, src_v, dst_v, bufs, gsem,
             ssem, acc, cid, row0, cnt=None):
    pltpu.sync_copy(z, acc.at[pl.ds(row0, ROWS_PT)])
    plsc.subcore_barrier()

    @pl.when(cid == 0)
    def _():
        _edge_steps(tab_c0, src_v, dst_v, bufs, gsem, ssem, acc, cnt)

    @pl.when(cid == 1)
    def _():
        _edge_steps(tab_c1, src_v, dst_v, bufs, gsem, ssem, acc, cnt)

    plsc.subcore_barrier()

    @pl.when(cid == 0)
    def _():
        pltpu.sync_copy(acc.at[pl.ds(row0, ROWS_PT)], out_c0.at[pl.ds(row0, ROWS_PT)])

    @pl.when(cid == 1)
    def _():
        pltpu.sync_copy(acc.at[pl.ds(row0, ROWS_PT)], out_c1.at[pl.ds(row0, ROWS_PT)])

    plsc.subcore_barrier()


_SC_SCRATCH = (
    pltpu.VMEM((NCHUNK, CH), jnp.int32),
    pltpu.VMEM((NCHUNK, CH), jnp.int32),
) + tuple(
    pltpu.VMEM((CH, Q), jnp.float32) for _ in range(NBUF)
) + (
    pltpu.VMEM_SHARED((NPAD, Q), jnp.float32),
    pltpu.SemaphoreType.DMA,
    pltpu.SemaphoreType.DMA,
)


@functools.partial(
    pl.kernel,
    out_type=tuple(
        jax.ShapeDtypeStruct((NPAD, Q), jnp.float32) for _ in range(2)
    ) + tuple(
        jax.ShapeDtypeStruct((NPAD, CW), jnp.float32) for _ in range(2)
    ),
    mesh=_sc_mesh,
    compiler_params=_sc_params,
    scratch_types=_SC_SCRATCH + (
        pltpu.VMEM((CH, CW), jnp.float32),
        pltpu.VMEM_SHARED((NPAD, CW), jnp.float32),
        pltpu.SemaphoreType.DMA,
    ),
)
def _sc_l1a(t0, t2, srcg, dstg, z, zc, ones_hbm,
            a0, a2, c0, c1,
            src_v, dst_v, b0, b1, b2, b3, b4, acc, gsem, ssem,
            ones_v, cntacc, csem):
    bufs = (b0, b1, b2, b3, b4)
    cid = lax.axis_index("c")
    sid = lax.axis_index("s")
    row0 = sid * ROWS_PT
    pltpu.sync_copy(srcg.at[sid], src_v)
    pltpu.sync_copy(dstg.at[sid], dst_v)
    pltpu.sync_copy(ones_hbm, ones_v)
    pltpu.sync_copy(zc, cntacc.at[pl.ds(row0, ROWS_PT)])
    cnt = (ones_v, cntacc, csem, cid)
    _sc_pass(t0, t2, a0, a2, z, src_v, dst_v, bufs, gsem, ssem, acc, cid,
             row0, cnt)

    @pl.when(cid == 0)
    def _():
        pltpu.sync_copy(cntacc.at[pl.ds(row0, ROWS_PT)],
                        c0.at[pl.ds(row0, ROWS_PT)])

    @pl.when(cid == 1)
    def _():
        pltpu.sync_copy(cntacc.at[pl.ds(row0, ROWS_PT)],
                        c1.at[pl.ds(row0, ROWS_PT)])


@functools.partial(
    pl.kernel,
    out_type=tuple(
        jax.ShapeDtypeStruct((NPAD, Q), jnp.float32) for _ in range(2)
    ),
    mesh=_sc_mesh,
    compiler_params=_sc_params,
    scratch_types=_SC_SCRATCH,
)
def _sc_half(t1, t3, srcg, dstg, z,
             a1, a3, src_v, dst_v, b0, b1, b2, b3, b4, acc, gsem, ssem):
    bufs = (b0, b1, b2, b3, b4)
    cid = lax.axis_index("c")
    sid = lax.axis_index("s")
    row0 = sid * ROWS_PT
    pltpu.sync_copy(srcg.at[sid], src_v)
    pltpu.sync_copy(dstg.at[sid], dst_v)
    _sc_pass(t1, t3, a1, a3, z, src_v, dst_v, bufs, gsem, ssem, acc, cid,
             row0)


BN = 2000


def _hpart_body(a0, a2, c0, c1, x0, x2, w1q0, w1q2, hA, rdeg):
    rd = 1.0 / (c0[:, 0:1] + c1[:, 0:1] + 1.0)
    t0 = ((a0[...] + x0[...]) * rd).astype(jnp.bfloat16)
    t2 = ((a2[...] + x2[...]) * rd).astype(jnp.bfloat16)
    hA[...] = (
        jnp.dot(t0, w1q0[...].astype(jnp.bfloat16),
                preferred_element_type=jnp.float32)
        + jnp.dot(t2, w1q2[...].astype(jnp.bfloat16),
                  preferred_element_type=jnp.float32)
    )
    rdeg[...] = rd


_hpart = pl.pallas_call(
    _hpart_body,
    grid=(N_NODES // BN,),
    in_specs=[
        pl.BlockSpec((BN, Q), lambda i: (i, 0)),
        pl.BlockSpec((BN, Q), lambda i: (i, 0)),
        pl.BlockSpec((BN, CW), lambda i: (i, 0)),
        pl.BlockSpec((BN, CW), lambda i: (i, 0)),
        pl.BlockSpec((BN, Q), lambda i: (i, 0)),
        pl.BlockSpec((BN, Q), lambda i: (i, 0)),
        pl.BlockSpec((Q, D_HID), lambda i: (0, 0)),
        pl.BlockSpec((Q, D_HID), lambda i: (0, 0)),
    ],
    out_specs=[
        pl.BlockSpec((BN, D_HID), lambda i: (i, 0)),
        pl.BlockSpec((BN, 1), lambda i: (i, 0)),
    ],
    out_shape=[
        jax.ShapeDtypeStruct((N_NODES, D_HID), jnp.float32),
        jax.ShapeDtypeStruct((N_NODES, 1), jnp.float32),
    ],
)


def _mpart_body(hA, a1, a3, x1, x3, rdeg, w1q1, w1q3, w2, m0, m1, m2, m3):
    rd = rdeg[...]
    t1 = ((a1[...] + x1[...]) * rd).astype(jnp.bfloat16)
    t3 = ((a3[...] + x3[...]) * rd).astype(jnp.bfloat16)
    h = hA[...] + jnp.dot(t1, w1q1[...].astype(jnp.bfloat16),
                          preferred_element_type=jnp.float32)
    h = h + jnp.dot(t3, w1q3[...].astype(jnp.bfloat16),
                    preferred_element_type=jnp.float32)
    hb = jnp.maximum(h, 0.0).astype(jnp.bfloat16)
    m = jnp.dot(hb, w2[...].astype(jnp.bfloat16),
                preferred_element_type=jnp.float32)
    m0[...] = m[:, 0 * Q:1 * Q]
    m1[...] = m[:, 1 * Q:2 * Q]
    m2[...] = m[:, 2 * Q:3 * Q]
    m3[...] = m[:, 3 * Q:4 * Q]


_mpart = pl.pallas_call(
    _mpart_body,
    grid=(N_NODES // BN,),
    in_specs=[
        pl.BlockSpec((BN, D_HID), lambda i: (i, 0)),
        pl.BlockSpec((BN, Q), lambda i: (i, 0)),
        pl.BlockSpec((BN, Q), lambda i: (i, 0)),
        pl.BlockSpec((BN, Q), lambda i: (i, 0)),
        pl.BlockSpec((BN, Q), lambda i: (i, 0)),
        pl.BlockSpec((BN, 1), lambda i: (i, 0)),
        pl.BlockSpec((Q, D_HID), lambda i: (0, 0)),
        pl.BlockSpec((Q, D_HID), lambda i: (0, 0)),
        pl.BlockSpec((D_HID, D_OUT), lambda i: (0, 0)),
    ],
    out_specs=[
        pl.BlockSpec((BN, Q), lambda i: (i, 0)),
        pl.BlockSpec((BN, Q), lambda i: (i, 0)),
        pl.BlockSpec((BN, Q), lambda i: (i, 0)),
        pl.BlockSpec((BN, Q), lambda i: (i, 0)),
    ],
    out_shape=[
        jax.ShapeDtypeStruct((N_NODES, Q), jnp.float32) for _ in range(4)
    ],
)


def _fin_body(ga, gb, ma, mb, rdeg, oa, ob):
    rd = rdeg[...]
    oa[...] = (ga[...] + ma[...]) * rd
    ob[...] = (gb[...] + mb[...]) * rd


_fin = pl.pallas_call(
    _fin_body,
    grid=(N_NODES // BN,),
    in_specs=[
        pl.BlockSpec((BN, Q), lambda i: (i, 0)),
        pl.BlockSpec((BN, Q), lambda i: (i, 0)),
        pl.BlockSpec((BN, Q), lambda i: (i, 0)),
        pl.BlockSpec((BN, Q), lambda i: (i, 0)),
        pl.BlockSpec((BN, 1), lambda i: (i, 0)),
    ],
    out_specs=[
        pl.BlockSpec((BN, Q), lambda i: (i, 0)),
        pl.BlockSpec((BN, Q), lambda i: (i, 0)),
    ],
    out_shape=[
        jax.ShapeDtypeStruct((N_NODES, Q), jnp.float32) for _ in range(2)
    ],
)


def kernel(x, edge_index, W1, W2):
    ei = edge_index.astype(jnp.int32)
    srcg = ei[0].reshape(NS, NCHUNK, CH)
    dstg = ei[1].reshape(NS, NCHUNK, CH)
    t0 = x[:, 0 * Q:1 * Q]
    t1 = x[:, 1 * Q:2 * Q]
    t2 = x[:, 2 * Q:3 * Q]
    t3 = x[:, 3 * Q:4 * Q]
    zq = jnp.zeros((ROWS_PT, Q), jnp.float32)
    zc = jnp.zeros((ROWS_PT, CW), jnp.float32)
    ones_hbm = jnp.ones((CH, CW), jnp.float32)

    a0, a2, c0, c1 = _sc_l1a(t0, t2, srcg, dstg, zq, zc, ones_hbm)
    a1, a3 = _sc_half(t1, t3, srcg, dstg, zq)
    hA, rdeg = _hpart(a0, a2, c0, c1, t0, t2, W1[0 * Q:1 * Q], W1[2 * Q:3 * Q])
    m0, m1, m2, m3 = _mpart(hA, a1, a3, t1, t3, rdeg,
                            W1[1 * Q:2 * Q], W1[3 * Q:4 * Q], W2)
    g0, g2 = _sc_half(m0, m2, srcg, dstg, zq)
    g1, g3 = _sc_half(m1, m3, srcg, dstg, zq)
    o0, o2 = _fin(g0, g2, m0, m2, rdeg)
    o1, o3 = _fin(g1, g3, m1, m3, rdeg)
    return jnp.concatenate([o0, o1, o2, o3], axis=1)


# speedup vs baseline: 1.1194x; 1.1194x over previous
import functools

import jax
import jax.numpy as jnp
from jax import lax
from jax.experimental import pallas as pl
from jax.experimental.pallas import tpu as pltpu
from jax.experimental.pallas import tpu_sc as plsc

N_NODES = 10000
N_EDGES = 160000
D_IN = 256
D_HID = 512
D_OUT = 256

NC = 2
NS = 16
Q = 64
CW = 16
CH = 125
EPT = N_EDGES // NS
NCHUNK = EPT // CH
NPAD = 10240
ROWS_PT = NPAD // NS

_sc_mesh = plsc.VectorSubcoreMesh(
    core_axis_name="c", subcore_axis_name="s", num_cores=NC, num_subcores=NS
)
_sc_params = pltpu.CompilerParams(use_tc_tiling_on_sc=False)


def _fire_gather(tab, src_v, buf, gsem, c):
    pltpu.async_copy(tab.at[src_v.at[c]], buf, gsem)


def _drain_gather(tab, src_v, buf, gsem, c):
    pltpu.make_async_copy(tab.at[src_v.at[c]], buf, gsem).wait()


def _fire_scatter(acc, dst_v, buf, ssem, c):
    pltpu.async_copy(buf, acc.at[dst_v.at[c]], ssem, add=True)


def _drain_scatter(acc, dst_v, buf, ssem, c):
    pltpu.make_async_copy(buf, acc.at[dst_v.at[c]], ssem).wait()


NBUF = 5


def _edge_steps(tab, src_v, dst_v, bufs, gsem, ssem, acc, cnt=None):
    def fire_cnt(c):
        if cnt is None:
            return
        ones_v, cntacc, csem, cid = cnt

        @pl.when(cid == (c % 2))
        def _():
            pltpu.async_copy(ones_v, cntacc.at[dst_v.at[c]], csem, add=True)

    for j in range(3):
        _fire_gather(tab, src_v, bufs[j], gsem, j)
    for j in range(NBUF):
        _drain_gather(tab, src_v, bufs[j], gsem, j)
        _fire_scatter(acc, dst_v, bufs[j], ssem, j)
        fire_cnt(j)
        if j >= 2:
            _drain_scatter(acc, dst_v, bufs[j - 2], ssem, j - 2)
        _fire_gather(tab, src_v, bufs[(j + 3) % NBUF], gsem, j + 3)

    def body(i, carry):
        c0 = NBUF * i
        for j in range(NBUF):
            c = c0 + j
            _drain_gather(tab, src_v, bufs[j], gsem, c)
            _fire_scatter(acc, dst_v, bufs[j], ssem, c)
            fire_cnt(c)
            _drain_scatter(acc, dst_v, bufs[(j + 3) % NBUF], ssem, c - 2)
            cg = jnp.minimum(c + 3, NCHUNK - 1)
            _fire_gather(tab, src_v, bufs[(j + 3) % NBUF], gsem, cg)
        return carry

    lax.fori_loop(1, NCHUNK // NBUF, body, 0)

    for j in range(2):
        c = NCHUNK - 2 + j
        _drain_scatter(acc, dst_v, bufs[c % NBUF], ssem, c)
    for j in range(3):
        _drain_gather(tab, src_v, bufs[j], gsem, NCHUNK - 1)
    if cnt is not None:
        ones_v, cntacc, csem, cid = cnt

        def cdrain(t, carry):
            pltpu.make_async_copy(ones_v, cntacc.at[dst_v.at[0]], csem).wait()
            return carry

        lax.fori_loop(0, NCHUNK // 2, cdrain, 0)


def _sc_pass(tab_c0, tab_c1, out_c0, out_c1, z, src_v, dst_v, bufs, gsem,
             ssem, acc, cid, row0, cnt=None):
    pltpu.sync_copy(z, acc.at[pl.ds(row0, ROWS_PT)])
    plsc.subcore_barrier()

    @pl.when(cid == 0)
    def _():
        _edge_steps(tab_c0, src_v, dst_v, bufs, gsem, ssem, acc, cnt)

    @pl.when(cid == 1)
    def _():
        _edge_steps(tab_c1, src_v, dst_v, bufs, gsem, ssem, acc, cnt)

    plsc.subcore_barrier()

    @pl.when(cid == 0)
    def _():
        pltpu.sync_copy(acc.at[pl.ds(row0, ROWS_PT)], out_c0.at[pl.ds(row0, ROWS_PT)])

    @pl.when(cid == 1)
    def _():
        pltpu.sync_copy(acc.at[pl.ds(row0, ROWS_PT)], out_c1.at[pl.ds(row0, ROWS_PT)])

    plsc.subcore_barrier()


_SC_SCRATCH = (
    pltpu.VMEM((NCHUNK, CH), jnp.int32),
    pltpu.VMEM((NCHUNK, CH), jnp.int32),
) + tuple(
    pltpu.VMEM((CH, Q), jnp.float32) for _ in range(NBUF)
) + (
    pltpu.VMEM_SHARED((NPAD, Q), jnp.float32),
    pltpu.SemaphoreType.DMA,
    pltpu.SemaphoreType.DMA,
)


@functools.partial(
    pl.kernel,
    out_type=tuple(
        jax.ShapeDtypeStruct((NPAD, Q), jnp.float32) for _ in range(4)
    ) + tuple(
        jax.ShapeDtypeStruct((NPAD, CW), jnp.float32) for _ in range(2)
    ),
    mesh=_sc_mesh,
    compiler_params=_sc_params,
    scratch_types=_SC_SCRATCH + (
        pltpu.VMEM((CH, CW), jnp.float32),
        pltpu.VMEM_SHARED((NPAD, CW), jnp.float32),
        pltpu.SemaphoreType.DMA,
    ),
)
def _sc_agg_l1(t0, t1, t2, t3, srcg, dstg, z, zc, ones_hbm,
               a0, a1, a2, a3, c0, c1,
               src_v, dst_v, b0, b1, b2, b3, b4, acc, gsem, ssem,
               ones_v, cntacc, csem):
    bufs = (b0, b1, b2, b3, b4)
    cid = lax.axis_index("c")
    sid = lax.axis_index("s")
    row0 = sid * ROWS_PT
    pltpu.sync_copy(srcg.at[sid], src_v)
    pltpu.sync_copy(dstg.at[sid], dst_v)
    pltpu.sync_copy(ones_hbm, ones_v)
    pltpu.sync_copy(zc, cntacc.at[pl.ds(row0, ROWS_PT)])
    cnt = (ones_v, cntacc, csem, cid)
    _sc_pass(t0, t2, a0, a2, z, src_v, dst_v, bufs, gsem, ssem, acc, cid,
             row0, cnt)
    _sc_pass(t1, t3, a1, a3, z, src_v, dst_v, bufs, gsem, ssem, acc, cid,
             row0)

    @pl.when(cid == 0)
    def _():
        pltpu.sync_copy(cntacc.at[pl.ds(row0, ROWS_PT)],
                        c0.at[pl.ds(row0, ROWS_PT)])

    @pl.when(cid == 1)
    def _():
        pltpu.sync_copy(cntacc.at[pl.ds(row0, ROWS_PT)],
                        c1.at[pl.ds(row0, ROWS_PT)])


@functools.partial(
    pl.kernel,
    out_type=tuple(
        jax.ShapeDtypeStruct((NPAD, Q), jnp.float32) for _ in range(4)
    ),
    mesh=_sc_mesh,
    compiler_params=_sc_params,
    scratch_types=_SC_SCRATCH,
)
def _sc_agg_l2(t0, t1, t2, t3, srcg, dstg, z,
               a0, a1, a2, a3, src_v, dst_v, b0, b1, b2, b3, b4,
               acc, gsem, ssem):
    bufs = (b0, b1, b2, b3, b4)
    cid = lax.axis_index("c")
    sid = lax.axis_index("s")
    row0 = sid * ROWS_PT
    pltpu.sync_copy(srcg.at[sid], src_v)
    pltpu.sync_copy(dstg.at[sid], dst_v)
    _sc_pass(t0, t2, a0, a2, z, src_v, dst_v, bufs, gsem, ssem, acc, cid,
             row0)
    _sc_pass(t1, t3, a1, a3, z, src_v, dst_v, bufs, gsem, ssem, acc, cid,
             row0)


BN = 2000


def _dense_body(a0, a1, a2, a3, c0, c1, x, w1, w2, m0, m1, m2, m3):
    deg = c0[:, 0:1] + c1[:, 0:1] + 1.0
    agg = jnp.concatenate(
        [a0[...], a1[...], a2[...], a3[...]], axis=1
    ) + x[...]
    t = (agg / deg).astype(jnp.bfloat16)
    h = jnp.maximum(
        jnp.dot(t, w1[...].astype(jnp.bfloat16),
                preferred_element_type=jnp.float32), 0.0).astype(jnp.bfloat16)
    m = jnp.dot(h, w2[...].astype(jnp.bfloat16),
                preferred_element_type=jnp.float32)
    m0[...] = m[:, 0 * Q:1 * Q]
    m1[...] = m[:, 1 * Q:2 * Q]
    m2[...] = m[:, 2 * Q:3 * Q]
    m3[...] = m[:, 3 * Q:4 * Q]


_dense = pl.pallas_call(
    _dense_body,
    grid=(N_NODES // BN,),
    in_specs=[
        pl.BlockSpec((BN, Q), lambda i: (i, 0)),
        pl.BlockSpec((BN, Q), lambda i: (i, 0)),
        pl.BlockSpec((BN, Q), lambda i: (i, 0)),
        pl.BlockSpec((BN, Q), lambda i: (i, 0)),
        pl.BlockSpec((BN, CW), lambda i: (i, 0)),
        pl.BlockSpec((BN, CW), lambda i: (i, 0)),
        pl.BlockSpec((BN, D_IN), lambda i: (i, 0)),
        pl.BlockSpec((D_IN, D_HID), lambda i: (0, 0)),
        pl.BlockSpec((D_HID, D_OUT), lambda i: (0, 0)),
    ],
    out_specs=[
        pl.BlockSpec((BN, Q), lambda i: (i, 0)),
        pl.BlockSpec((BN, Q), lambda i: (i, 0)),
        pl.BlockSpec((BN, Q), lambda i: (i, 0)),
        pl.BlockSpec((BN, Q), lambda i: (i, 0)),
    ],
    out_shape=[
        jax.ShapeDtypeStruct((N_NODES, Q), jnp.float32) for _ in range(4)
    ],
)


def _final_body(g0, g1, g2, g3, m0, m1, m2, m3, c0, c1, out):
    deg = c0[:, 0:1] + c1[:, 0:1] + 1.0
    out[...] = jnp.concatenate(
        [g0[...] + m0[...], g1[...] + m1[...], g2[...] + m2[...], g3[...] + m3[...]],
        axis=1,
    ) / deg


_final = pl.pallas_call(
    _final_body,
    grid=(N_NODES // BN,),
    in_specs=[
        pl.BlockSpec((BN, Q), lambda i: (i, 0)),
        pl.BlockSpec((BN, Q), lambda i: (i, 0)),
        pl.BlockSpec((BN, Q), lambda i: (i, 0)),
        pl.BlockSpec((BN, Q), lambda i: (i, 0)),
        pl.BlockSpec((BN, Q), lambda i: (i, 0)),
        pl.BlockSpec((BN, Q), lambda i: (i, 0)),
        pl.BlockSpec((BN, Q), lambda i: (i, 0)),
        pl.BlockSpec((BN, Q), lambda i: (i, 0)),
        pl.BlockSpec((BN, CW), lambda i: (i, 0)),
        pl.BlockSpec((BN, CW), lambda i: (i, 0)),
    ],
    out_specs=pl.BlockSpec((BN, D_OUT), lambda i: (i, 0)),
    out_shape=jax.ShapeDtypeStruct((N_NODES, D_OUT), jnp.float32),
)


def kernel(x, edge_index, W1, W2):
    ei = edge_index.astype(jnp.int32)
    srcg = ei[0].reshape(NS, NCHUNK, CH)
    dstg = ei[1].reshape(NS, NCHUNK, CH)
    t0 = x[:, 0 * Q:1 * Q]
    t1 = x[:, 1 * Q:2 * Q]
    t2 = x[:, 2 * Q:3 * Q]
    t3 = x[:, 3 * Q:4 * Q]
    zq = jnp.zeros((ROWS_PT, Q), jnp.float32)
    zc = jnp.zeros((ROWS_PT, CW), jnp.float32)
    ones_hbm = jnp.ones((CH, CW), jnp.float32)

    a0, a1, a2, a3, c0, c1 = _sc_agg_l1(t0, t1, t2, t3, srcg, dstg, zq, zc,
                                        ones_hbm)
    m0, m1, m2, m3 = _dense(a0, a1, a2, a3, c0, c1, x, W1, W2)
    g0, g1, g2, g3 = _sc_agg_l2(m0, m1, m2, m3, srcg, dstg, zq)
    return _final(g0, g1, g2, g3, m0, m1, m2, m3, c0, c1)
